# baseline (speedup 1.0000x reference)
import functools

import jax
import jax.numpy as jnp
from jax import lax
from jax.experimental import pallas as pl
from jax.experimental.pallas import tpu as pltpu
from jax.experimental.pallas import tpu_sc as plsc

N = 10000
E = 320000
D = 128

NUM_CORES = 2
NUM_SUBCORES = 16
NUM_TILES = NUM_CORES * NUM_SUBCORES
CHUNK = 80
CHUNKS_PER_TILE = 128
E_PAD = NUM_TILES * CHUNKS_PER_TILE * CHUNK
IDX_GRP = 16
NUM_GRPS = CHUNKS_PER_TILE // IDX_GRP
ROWS_PER_TILE = 640
N_PAD = ROWS_PER_TILE * NUM_SUBCORES
WB_BLKS = ROWS_PER_TILE // CHUNK


def _sc_aggregate():
    mesh = plsc.VectorSubcoreMesh(core_axis_name="c", subcore_axis_name="s")

    @functools.partial(
        pl.kernel,
        mesh=mesh,
        out_type=jax.ShapeDtypeStruct((NUM_CORES, NUM_SUBCORES, ROWS_PER_TILE, D),
                                      jnp.float32),
        scratch_types=[
            pltpu.VMEM((IDX_GRP, CHUNK), jnp.int32),
            pltpu.VMEM((IDX_GRP, CHUNK), jnp.int32),
            pltpu.VMEM((CHUNK, D), jnp.float32),
            pltpu.VMEM((CHUNK, D), jnp.float32),
            pltpu.VMEM_SHARED((N_PAD, D), jnp.float32),
            pltpu.SemaphoreType.DMA,
            pltpu.SemaphoreType.DMA,
            pltpu.SemaphoreType.DMA,
            pltpu.SemaphoreType.DMA,
        ],
    )
    def sc_agg(src_hbm, dst_hbm, x_hbm, zeros_hbm, out_hbm,
               src_v, dst_v, rows0, rows1, acc_sh, gs0, gs1, ss0, ss1):
        cid = lax.axis_index("c")
        sid = lax.axis_index("s")
        wid = cid * NUM_SUBCORES + sid
        row_off = pl.multiple_of(sid * ROWS_PER_TILE, 8)

        pltpu.sync_copy(zeros_hbm, rows0)
        for j in range(WB_BLKS):
            pltpu.sync_copy(rows0, acc_sh.at[pl.ds(row_off + j * CHUNK, CHUNK)])
        plsc.subcore_barrier()

        bufs = (rows0, rows1)
        gsems = (gs0, gs1)
        ssems = (ss0, ss1)

        def sg(j, k):
            pltpu.async_copy(x_hbm.at[src_v.at[j]], bufs[k], gsems[k])

        def wg(k):
            pltpu.make_async_copy(zeros_hbm, bufs[k], gsems[k]).wait()

        def st(j, k):
            pltpu.async_copy(bufs[k], acc_sh.at[dst_v.at[j]], ssems[k], add=True)

        def ws(k):
            pltpu.make_async_copy(zeros_hbm, bufs[k], ssems[k]).wait()

        def group(grp, carry):
            goff = pl.multiple_of(grp * IDX_GRP, 8)
            pltpu.sync_copy(src_hbm.at[wid, pl.ds(goff, IDX_GRP)], src_v)
            pltpu.sync_copy(dst_hbm.at[wid, pl.ds(goff, IDX_GRP)], dst_v)

            sg(0, 0)
            sg(1, 1)

            def body(g, c):
                j = 2 * g
                wg(0)
                st(j, 0)
                wg(1)
                st(j + 1, 1)
                ws(0)
                sg(j + 2, 0)
                ws(1)
                sg(j + 3, 1)
                return c

            lax.fori_loop(0, IDX_GRP // 2 - 1, body, 0)
            last = IDX_GRP - 2
            wg(0)
            st(last, 0)
            wg(1)
            st(last + 1, 1)
            ws(0)
            ws(1)
            return carry

        lax.fori_loop(0, NUM_GRPS, group, 0)

        plsc.subcore_barrier()

        for j in range(WB_BLKS):
            pltpu.sync_copy(acc_sh.at[pl.ds(row_off + j * CHUNK, CHUNK)], rows0)
            pltpu.sync_copy(rows0, out_hbm.at[cid, sid, pl.ds(j * CHUNK, CHUNK)])

    return sc_agg


_SC_AGG = _sc_aggregate()


def _tc_fn(x_ref, a0_ref, a1_ref, w_ref, b_ref, s_ref, o_ref):
    h = s_ref[0, 0] * x_ref[...] + a0_ref[...] + a1_ref[...]
    o = jnp.dot(h, w_ref[...], preferred_element_type=jnp.float32) + b_ref[...]
    o_ref[...] = jnp.maximum(o, 0.0)


ROW_BLK = 1000


def _tc_dense(x, a0, a1, W, b2, scale):
    return pl.pallas_call(
        _tc_fn,
        grid=(N // ROW_BLK,),
        in_specs=[
            pl.BlockSpec((ROW_BLK, D), lambda i: (i, 0)),
            pl.BlockSpec((ROW_BLK, D), lambda i: (i, 0)),
            pl.BlockSpec((ROW_BLK, D), lambda i: (i, 0)),
            pl.BlockSpec((D, D), lambda i: (0, 0)),
            pl.BlockSpec((1, D), lambda i: (0, 0)),
            pl.BlockSpec(memory_space=pltpu.SMEM),
        ],
        out_specs=pl.BlockSpec((ROW_BLK, D), lambda i: (i, 0)),
        out_shape=jax.ShapeDtypeStruct((N, D), jnp.float32),
    )(x, a0, a1, W, b2, scale)


def kernel(x, edge_index, W, b, eps):
    pad = E_PAD - E
    src = jnp.concatenate([edge_index[0], jnp.zeros((pad,), jnp.int32)])
    dst = jnp.concatenate([edge_index[1], jnp.full((pad,), N, jnp.int32)])
    src = src.reshape(NUM_TILES, CHUNKS_PER_TILE, CHUNK)
    dst = dst.reshape(NUM_TILES, CHUNKS_PER_TILE, CHUNK)
    zeros = jnp.zeros((CHUNK, D), jnp.float32)
    partials = _SC_AGG(src, dst, x, zeros)
    partials = partials.reshape(NUM_CORES, N_PAD, D)[:, :N, :]
    scale = (1.0 + eps).astype(jnp.float32).reshape(1, 1)
    return _tc_dense(x, partials[0], partials[1], W, b.reshape(1, D), scale)


# speedup vs baseline: 2.6774x; 2.6774x over previous
import functools

import jax
import jax.numpy as jnp
from jax import lax
from jax.experimental import pallas as pl
from jax.experimental.pallas import tpu as pltpu
from jax.experimental.pallas import tpu_sc as plsc

N = 10000
E = 320000
D = 128

NUM_CORES = 2
NUM_SUBCORES = 16
NUM_TILES = NUM_CORES * NUM_SUBCORES
CHUNK = 80
CHUNKS_PER_TILE = 128
E_PAD = NUM_TILES * CHUNKS_PER_TILE * CHUNK
IDX_GRP = 16
NUM_GRPS = CHUNKS_PER_TILE // IDX_GRP
ROWS_PER_TILE = 640
N_PAD = ROWS_PER_TILE * NUM_SUBCORES
WB_BLKS = ROWS_PER_TILE // CHUNK


def _sc_aggregate():
    mesh = plsc.VectorSubcoreMesh(core_axis_name="c", subcore_axis_name="s")

    @functools.partial(
        pl.kernel,
        mesh=mesh,
        out_type=jax.ShapeDtypeStruct((NUM_CORES, NUM_SUBCORES, ROWS_PER_TILE, D),
                                      jnp.float32),
        scratch_types=[
            pltpu.VMEM((IDX_GRP, CHUNK), jnp.int32),
            pltpu.VMEM((IDX_GRP, CHUNK), jnp.int32),
            pltpu.VMEM((CHUNK, D), jnp.float32),
            pltpu.VMEM((CHUNK, D), jnp.float32),
            pltpu.VMEM_SHARED((N_PAD, D), jnp.float32),
            pltpu.SemaphoreType.DMA,
            pltpu.SemaphoreType.DMA,
            pltpu.SemaphoreType.DMA,
            pltpu.SemaphoreType.DMA,
        ],
    )
    def sc_agg(src_hbm, dst_hbm, x_hbm, zeros_hbm, out_hbm,
               src_v, dst_v, rows0, rows1, acc_sh, gs0, gs1, ss0, ss1):
        cid = lax.axis_index("c")
        sid = lax.axis_index("s")
        wid = cid * NUM_SUBCORES + sid
        row_off = pl.multiple_of(sid * ROWS_PER_TILE, 8)

        pltpu.sync_copy(zeros_hbm, rows0)
        for j in range(WB_BLKS):
            pltpu.sync_copy(rows0, acc_sh.at[pl.ds(row_off + j * CHUNK, CHUNK)])
        plsc.subcore_barrier()

        bufs = (rows0, rows1)
        gsems = (gs0, gs1)
        ssems = (ss0, ss1)

        def sg(j, k):
            pltpu.async_copy(x_hbm.at[src_v.at[j]], bufs[k], gsems[k])

        def wg(k):
            pltpu.make_async_copy(zeros_hbm, bufs[k], gsems[k]).wait()

        def st(j, k):
            pltpu.async_copy(bufs[k], acc_sh.at[dst_v.at[j]], ssems[k], add=True)

        def ws(k):
            pltpu.make_async_copy(zeros_hbm, bufs[k], ssems[k]).wait()

        def group(grp, carry):
            goff = pl.multiple_of(grp * IDX_GRP, 8)
            pltpu.sync_copy(src_hbm.at[wid, pl.ds(goff, IDX_GRP)], src_v)
            pltpu.sync_copy(dst_hbm.at[wid, pl.ds(goff, IDX_GRP)], dst_v)

            sg(0, 0)
            sg(1, 1)

            def body(g, c):
                j = 2 * g
                wg(0)
                st(j, 0)
                wg(1)
                st(j + 1, 1)
                ws(0)
                sg(j + 2, 0)
                ws(1)
                sg(j + 3, 1)
                return c

            lax.fori_loop(0, IDX_GRP // 2 - 1, body, 0)
            last = IDX_GRP - 2
            wg(0)
            st(last, 0)
            wg(1)
            st(last + 1, 1)
            ws(0)
            ws(1)
            return carry

        lax.fori_loop(0, NUM_GRPS, group, 0)

        plsc.subcore_barrier()

        for j in range(WB_BLKS):
            pltpu.sync_copy(acc_sh.at[pl.ds(row_off + j * CHUNK, CHUNK)], rows0)
            pltpu.sync_copy(rows0, out_hbm.at[cid, sid, pl.ds(j * CHUNK, CHUNK)])

    return sc_agg


_SC_AGG = _sc_aggregate()


def _tc_fn(x_ref, a0_ref, a1_ref, w_ref, b_ref, s_ref, o_ref):
    h = s_ref[0, 0] * x_ref[...] + a0_ref[...] + a1_ref[...]
    o = jnp.dot(h, w_ref[...], preferred_element_type=jnp.float32) + b_ref[...]
    o_ref[...] = jnp.maximum(o, 0.0)


ROW_BLK = 1000


def _tc_dense(x, a0, a1, W, b2, scale):
    return pl.pallas_call(
        _tc_fn,
        grid=(N // ROW_BLK,),
        in_specs=[
            pl.BlockSpec((ROW_BLK, D), lambda i: (i, 0)),
            pl.BlockSpec((ROW_BLK, D), lambda i: (i, 0)),
            pl.BlockSpec((ROW_BLK, D), lambda i: (i, 0)),
            pl.BlockSpec((D, D), lambda i: (0, 0)),
            pl.BlockSpec((1, D), lambda i: (0, 0)),
            pl.BlockSpec(memory_space=pltpu.SMEM),
        ],
        out_specs=pl.BlockSpec((ROW_BLK, D), lambda i: (i, 0)),
        out_shape=jax.ShapeDtypeStruct((N, D), jnp.float32),
    )(x, a0, a1, W, b2, scale)


def kernel(x, edge_index, W, b, eps):
    pad = E_PAD - E
    pad_src = jnp.arange(pad, dtype=jnp.int32) % N
    pad_dst = N + (jnp.arange(pad, dtype=jnp.int32) % (N_PAD - N))
    src = jnp.concatenate([edge_index[0], pad_src])
    dst = jnp.concatenate([edge_index[1], pad_dst])
    src = src.reshape(NUM_TILES, CHUNKS_PER_TILE, CHUNK)
    dst = dst.reshape(NUM_TILES, CHUNKS_PER_TILE, CHUNK)
    zeros = jnp.zeros((CHUNK, D), jnp.float32)
    partials = _SC_AGG(src, dst, x, zeros)
    partials = partials.reshape(NUM_CORES, N_PAD, D)[:, :N, :]
    scale = (1.0 + eps).astype(jnp.float32).reshape(1, 1)
    return _tc_dense(x, partials[0], partials[1], W, b.reshape(1, D), scale)


# speedup vs baseline: 2.9262x; 1.0929x over previous
import functools

import jax
import jax.numpy as jnp
from jax import lax
from jax.experimental import pallas as pl
from jax.experimental.pallas import tpu as pltpu
from jax.experimental.pallas import tpu_sc as plsc

N = 10000
E = 320000
D = 128

NUM_CORES = 2
NUM_SUBCORES = 16
NUM_TILES = NUM_CORES * NUM_SUBCORES
CHUNK = 128
CHUNKS_PER_TILE = 80
E_PAD = NUM_TILES * CHUNKS_PER_TILE * CHUNK
IDX_GRP = 16
NUM_GRPS = CHUNKS_PER_TILE // IDX_GRP
ROWS_PER_TILE = 640
N_PAD = ROWS_PER_TILE * NUM_SUBCORES
WB_BLKS = ROWS_PER_TILE // CHUNK


def _sc_aggregate():
    mesh = plsc.VectorSubcoreMesh(core_axis_name="c", subcore_axis_name="s")

    @functools.partial(
        pl.kernel,
        mesh=mesh,
        out_type=jax.ShapeDtypeStruct((NUM_CORES, NUM_SUBCORES, ROWS_PER_TILE, D),
                                      jnp.float32),
        scratch_types=[
            pltpu.VMEM((IDX_GRP, CHUNK), jnp.int32),
            pltpu.VMEM((IDX_GRP, CHUNK), jnp.int32),
            pltpu.VMEM((CHUNK, D), jnp.float32),
            pltpu.VMEM((CHUNK, D), jnp.float32),
            pltpu.VMEM_SHARED((N_PAD, D), jnp.float32),
            pltpu.SemaphoreType.DMA,
            pltpu.SemaphoreType.DMA,
            pltpu.SemaphoreType.DMA,
            pltpu.SemaphoreType.DMA,
        ],
    )
    def sc_agg(src_hbm, dst_hbm, x_hbm, zeros_hbm, out_hbm,
               src_v, dst_v, rows0, rows1, acc_sh, gs0, gs1, ss0, ss1):
        cid = lax.axis_index("c")
        sid = lax.axis_index("s")
        wid = cid * NUM_SUBCORES + sid
        row_off = pl.multiple_of(sid * ROWS_PER_TILE, 8)

        pltpu.sync_copy(zeros_hbm, rows0)
        for j in range(WB_BLKS):
            pltpu.sync_copy(rows0, acc_sh.at[pl.ds(row_off + j * CHUNK, CHUNK)])
        plsc.subcore_barrier()

        bufs = (rows0, rows1)
        gsems = (gs0, gs1)
        ssems = (ss0, ss1)

        def sg(j, k):
            pltpu.async_copy(x_hbm.at[src_v.at[j]], bufs[k], gsems[k])

        def wg(k):
            pltpu.make_async_copy(zeros_hbm, bufs[k], gsems[k]).wait()

        def st(j, k):
            pltpu.async_copy(bufs[k], acc_sh.at[dst_v.at[j]], ssems[k], add=True)

        def ws(k):
            pltpu.make_async_copy(zeros_hbm, bufs[k], ssems[k]).wait()

        def group(grp, carry):
            goff = pl.multiple_of(grp * IDX_GRP, 8)
            pltpu.sync_copy(src_hbm.at[wid, pl.ds(goff, IDX_GRP)], src_v)
            pltpu.sync_copy(dst_hbm.at[wid, pl.ds(goff, IDX_GRP)], dst_v)

            sg(0, 0)
            sg(1, 1)

            def body(g, c):
                j = 2 * g
                wg(0)
                st(j, 0)
                wg(1)
                st(j + 1, 1)
                ws(0)
                sg(j + 2, 0)
                ws(1)
                sg(j + 3, 1)
                return c

            lax.fori_loop(0, IDX_GRP // 2 - 1, body, 0)
            last = IDX_GRP - 2
            wg(0)
            st(last, 0)
            wg(1)
            st(last + 1, 1)
            ws(0)
            ws(1)
            return carry

        lax.fori_loop(0, NUM_GRPS, group, 0)

        plsc.subcore_barrier()

        for j in range(WB_BLKS):
            pltpu.sync_copy(acc_sh.at[pl.ds(row_off + j * CHUNK, CHUNK)], rows0)
            pltpu.sync_copy(rows0, out_hbm.at[cid, sid, pl.ds(j * CHUNK, CHUNK)])

    return sc_agg


_SC_AGG = _sc_aggregate()


def _tc_fn(x_ref, a0_ref, a1_ref, w_ref, b_ref, s_ref, o_ref):
    h = s_ref[0, 0] * x_ref[...] + a0_ref[...] + a1_ref[...]
    o = jnp.dot(h, w_ref[...], preferred_element_type=jnp.float32) + b_ref[...]
    o_ref[...] = jnp.maximum(o, 0.0)


ROW_BLK = 1000


def _tc_dense(x, a0, a1, W, b2, scale):
    return pl.pallas_call(
        _tc_fn,
        grid=(N // ROW_BLK,),
        in_specs=[
            pl.BlockSpec((ROW_BLK, D), lambda i: (i, 0)),
            pl.BlockSpec((ROW_BLK, D), lambda i: (i, 0)),
            pl.BlockSpec((ROW_BLK, D), lambda i: (i, 0)),
            pl.BlockSpec((D, D), lambda i: (0, 0)),
            pl.BlockSpec((1, D), lambda i: (0, 0)),
            pl.BlockSpec(memory_space=pltpu.SMEM),
        ],
        out_specs=pl.BlockSpec((ROW_BLK, D), lambda i: (i, 0)),
        out_shape=jax.ShapeDtypeStruct((N, D), jnp.float32),
    )(x, a0, a1, W, b2, scale)


def kernel(x, edge_index, W, b, eps):
    pad = E_PAD - E
    pad_src = jnp.arange(pad, dtype=jnp.int32) % N
    pad_dst = N + (jnp.arange(pad, dtype=jnp.int32) % (N_PAD - N))
    src = jnp.concatenate([edge_index[0], pad_src])
    dst = jnp.concatenate([edge_index[1], pad_dst])
    src = src.reshape(NUM_TILES, CHUNKS_PER_TILE, CHUNK)
    dst = dst.reshape(NUM_TILES, CHUNKS_PER_TILE, CHUNK)
    zeros = jnp.zeros((CHUNK, D), jnp.float32)
    partials = _SC_AGG(src, dst, x, zeros)
    partials = partials.reshape(NUM_CORES, N_PAD, D)[:, :N, :]
    scale = (1.0 + eps).astype(jnp.float32).reshape(1, 1)
    return _tc_dense(x, partials[0], partials[1], W, b.reshape(1, D), scale)


# speedup vs baseline: 3.0235x; 1.0332x over previous
import functools

import jax
import jax.numpy as jnp
from jax import lax
from jax.experimental import pallas as pl
from jax.experimental.pallas import tpu as pltpu
from jax.experimental.pallas import tpu_sc as plsc

N = 10000
E = 320000
D = 128

NUM_CORES = 2
NUM_SUBCORES = 16
NUM_TILES = NUM_CORES * NUM_SUBCORES
CHUNK = 128
CHUNKS_PER_TILE = 80
E_PAD = NUM_TILES * CHUNKS_PER_TILE * CHUNK
IDX_GRP = 16
NUM_GRPS = CHUNKS_PER_TILE // IDX_GRP
ROWS_PER_TILE = 640
N_PAD = ROWS_PER_TILE * NUM_SUBCORES
WB_BLKS = ROWS_PER_TILE // CHUNK


def _sc_aggregate():
    mesh = plsc.VectorSubcoreMesh(core_axis_name="c", subcore_axis_name="s")

    @functools.partial(
        pl.kernel,
        mesh=mesh,
        out_type=jax.ShapeDtypeStruct((NUM_CORES, NUM_SUBCORES, ROWS_PER_TILE, D),
                                      jnp.float32),
        scratch_types=[
            pltpu.VMEM((IDX_GRP, CHUNK), jnp.int32),
            pltpu.VMEM((IDX_GRP, CHUNK), jnp.int32),
            pltpu.VMEM((CHUNK, D), jnp.float32),
            pltpu.VMEM((CHUNK, D), jnp.float32),
            pltpu.VMEM_SHARED((N_PAD, D), jnp.float32),
            pltpu.SemaphoreType.DMA,
            pltpu.SemaphoreType.DMA,
            pltpu.SemaphoreType.DMA,
            pltpu.SemaphoreType.DMA,
        ],
    )
    def sc_agg(src_hbm, dst_hbm, x_hbm, zeros_hbm, out_hbm,
               src_v, dst_v, rows0, rows1, acc_sh, gs0, gs1, ss0, ss1):
        cid = lax.axis_index("c")
        sid = lax.axis_index("s")
        wid = cid * NUM_SUBCORES + sid
        row_off = pl.multiple_of(sid * ROWS_PER_TILE, 8)

        pltpu.sync_copy(zeros_hbm, rows0)
        for j in range(WB_BLKS):
            pltpu.sync_copy(rows0, acc_sh.at[pl.ds(row_off + j * CHUNK, CHUNK)])
        plsc.subcore_barrier()

        bufs = (rows0, rows1)
        gsems = (gs0, gs1)
        ssems = (ss0, ss1)

        def sg(j, k):
            pltpu.async_copy(x_hbm.at[src_v.at[j]], bufs[k], gsems[k])

        def wg(k):
            pltpu.make_async_copy(zeros_hbm, bufs[k], gsems[k]).wait()

        def st(j, k):
            pltpu.async_copy(bufs[k], acc_sh.at[dst_v.at[j]], ssems[k], add=True)

        def ws(k):
            pltpu.make_async_copy(zeros_hbm, bufs[k], ssems[k]).wait()

        def group(grp, carry):
            goff = pl.multiple_of(grp * IDX_GRP, 8)
            pltpu.sync_copy(src_hbm.at[wid, pl.ds(goff, IDX_GRP)], src_v)
            pltpu.sync_copy(dst_hbm.at[wid, pl.ds(goff, IDX_GRP)], dst_v)

            sg(0, 0)
            sg(1, 1)

            def body(g, c):
                j = 2 * g
                wg(0)
                st(j, 0)
                wg(1)
                st(j + 1, 1)
                ws(0)
                sg(j + 2, 0)
                ws(1)
                sg(j + 3, 1)
                return c

            lax.fori_loop(0, IDX_GRP // 2 - 1, body, 0)
            last = IDX_GRP - 2
            wg(0)
            st(last, 0)
            wg(1)
            st(last + 1, 1)
            ws(0)
            ws(1)
            return carry

        lax.fori_loop(0, NUM_GRPS, group, 0)

        plsc.subcore_barrier()

        for j in range(WB_BLKS):
            pltpu.sync_copy(acc_sh.at[pl.ds(row_off + j * CHUNK, CHUNK)], rows0)
            pltpu.sync_copy(rows0, out_hbm.at[cid, sid, pl.ds(j * CHUNK, CHUNK)])

    return sc_agg


_SC_AGG = _sc_aggregate()


def _tc_fn(x_ref, a_ref, w_ref, b_ref, s_ref, o_ref):
    h = s_ref[0, 0] * x_ref[...] + a_ref[0] + a_ref[1]
    o = jnp.dot(h, w_ref[...], preferred_element_type=jnp.float32) + b_ref[...]
    o_ref[...] = jnp.maximum(o, 0.0)


ROW_BLK = 1000


def _tc_dense(x, partials, W, b2, scale):
    return pl.pallas_call(
        _tc_fn,
        grid=(N // ROW_BLK,),
        in_specs=[
            pl.BlockSpec((ROW_BLK, D), lambda i: (i, 0)),
            pl.BlockSpec((NUM_CORES, ROW_BLK, D), lambda i: (0, i, 0)),
            pl.BlockSpec((D, D), lambda i: (0, 0)),
            pl.BlockSpec((1, D), lambda i: (0, 0)),
            pl.BlockSpec(memory_space=pltpu.SMEM),
        ],
        out_specs=pl.BlockSpec((ROW_BLK, D), lambda i: (i, 0)),
        out_shape=jax.ShapeDtypeStruct((N, D), jnp.float32),
    )(x, partials, W, b2, scale)


def kernel(x, edge_index, W, b, eps):
    pad = E_PAD - E
    pad_src = jnp.arange(pad, dtype=jnp.int32) % N
    pad_dst = N + (jnp.arange(pad, dtype=jnp.int32) % (N_PAD - N))
    src = jnp.concatenate([edge_index[0], pad_src])
    dst = jnp.concatenate([edge_index[1], pad_dst])
    src = src.reshape(NUM_TILES, CHUNKS_PER_TILE, CHUNK)
    dst = dst.reshape(NUM_TILES, CHUNKS_PER_TILE, CHUNK)
    zeros = jnp.zeros((CHUNK, D), jnp.float32)
    partials = _SC_AGG(src, dst, x, zeros)
    partials = partials.reshape(NUM_CORES, N_PAD, D)
    scale = (1.0 + eps).astype(jnp.float32).reshape(1, 1)
    return _tc_dense(x, partials, W, b.reshape(1, D), scale)


# speedup vs baseline: 3.0656x; 1.0139x over previous
import functools

import jax
import jax.numpy as jnp
from jax import lax
from jax.experimental import pallas as pl
from jax.experimental.pallas import tpu as pltpu
from jax.experimental.pallas import tpu_sc as plsc

N = 10000
E = 320000
D = 128

NUM_CORES = 2
NUM_SUBCORES = 16
NUM_TILES = NUM_CORES * NUM_SUBCORES
CHUNK = 128
CHUNKS_PER_TILE = 80
E_PAD = NUM_TILES * CHUNKS_PER_TILE * CHUNK
IDX_GRP = 16
NUM_GRPS = CHUNKS_PER_TILE // IDX_GRP
ROWS_PER_TILE = 640
N_PAD = ROWS_PER_TILE * NUM_SUBCORES
WB_BLKS = ROWS_PER_TILE // CHUNK


def _sc_aggregate():
    mesh = plsc.VectorSubcoreMesh(core_axis_name="c", subcore_axis_name="s")

    @functools.partial(
        pl.kernel,
        mesh=mesh,
        out_type=jax.ShapeDtypeStruct((NUM_CORES, NUM_SUBCORES, ROWS_PER_TILE, D),
                                      jnp.float32),
        scratch_types=[
            pltpu.VMEM((IDX_GRP, CHUNK), jnp.int32),
            pltpu.VMEM((IDX_GRP, CHUNK), jnp.int32),
            pltpu.VMEM((CHUNK, D), jnp.float32),
            pltpu.VMEM((CHUNK, D), jnp.float32),
            pltpu.VMEM_SHARED((N_PAD, D), jnp.float32),
            pltpu.SemaphoreType.DMA,
            pltpu.SemaphoreType.DMA,
            pltpu.SemaphoreType.DMA,
            pltpu.SemaphoreType.DMA,
        ],
    )
    def sc_agg(src_hbm, dst_hbm, x_hbm, zeros_hbm, out_hbm,
               src_v, dst_v, rows0, rows1, acc_sh, gs0, gs1, ss0, ss1):
        cid = lax.axis_index("c")
        sid = lax.axis_index("s")
        wid = cid * NUM_SUBCORES + sid
        row_off = pl.multiple_of(sid * ROWS_PER_TILE, 8)

        pltpu.sync_copy(zeros_hbm, rows0)
        for j in range(WB_BLKS):
            pltpu.sync_copy(rows0, acc_sh.at[pl.ds(row_off + j * CHUNK, CHUNK)])
        plsc.subcore_barrier()

        bufs = (rows0, rows1)
        gsems = (gs0, gs1)
        ssems = (ss0, ss1)

        def sg(j, k):
            pltpu.async_copy(x_hbm.at[src_v.at[j]], bufs[k], gsems[k])

        def wg(k):
            pltpu.make_async_copy(zeros_hbm, bufs[k], gsems[k]).wait()

        def st(j, k):
            pltpu.async_copy(bufs[k], acc_sh.at[dst_v.at[j]], ssems[k], add=True)

        def ws(k):
            pltpu.make_async_copy(zeros_hbm, bufs[k], ssems[k]).wait()

        def group(grp, carry):
            goff = pl.multiple_of(grp * IDX_GRP, 8)
            pltpu.sync_copy(src_hbm.at[wid, pl.ds(goff, IDX_GRP)], src_v)
            pltpu.sync_copy(dst_hbm.at[wid, pl.ds(goff, IDX_GRP)], dst_v)

            sg(0, 0)
            sg(1, 1)

            def body(g, c):
                j = 2 * g
                wg(0)
                st(j, 0)
                wg(1)
                st(j + 1, 1)
                ws(0)
                sg(j + 2, 0)
                ws(1)
                sg(j + 3, 1)
                return c

            lax.fori_loop(0, IDX_GRP // 2 - 1, body, 0)
            last = IDX_GRP - 2
            wg(0)
            st(last, 0)
            wg(1)
            st(last + 1, 1)
            ws(0)
            ws(1)
            return carry

        lax.fori_loop(0, NUM_GRPS, group, 0)

        plsc.subcore_barrier()

        for j in range(WB_BLKS):
            pltpu.sync_copy(acc_sh.at[pl.ds(row_off + j * CHUNK, CHUNK)], rows0)
            pltpu.sync_copy(rows0, out_hbm.at[cid, sid, pl.ds(j * CHUNK, CHUNK)])

    return sc_agg


_SC_AGG = _sc_aggregate()


def _tc_fn(x_ref, a_ref, w_ref, b_ref, s_ref, o_ref):
    h = s_ref[0, 0] * x_ref[...] + a_ref[0] + a_ref[1]
    o = jnp.dot(h, w_ref[...], preferred_element_type=jnp.float32) + b_ref[...]
    o_ref[...] = jnp.maximum(o, 0.0)


ROW_BLK = 2000


def _tc_dense(x, partials, W, b2, scale):
    return pl.pallas_call(
        _tc_fn,
        grid=(N // ROW_BLK,),
        in_specs=[
            pl.BlockSpec((ROW_BLK, D), lambda i: (i, 0)),
            pl.BlockSpec((NUM_CORES, ROW_BLK, D), lambda i: (0, i, 0)),
            pl.BlockSpec((D, D), lambda i: (0, 0)),
            pl.BlockSpec((1, D), lambda i: (0, 0)),
            pl.BlockSpec(memory_space=pltpu.SMEM),
        ],
        out_specs=pl.BlockSpec((ROW_BLK, D), lambda i: (i, 0)),
        out_shape=jax.ShapeDtypeStruct((N, D), jnp.float32),
    )(x, partials, W, b2, scale)


def kernel(x, edge_index, W, b, eps):
    pad = E_PAD - E
    pad_src = jnp.arange(pad, dtype=jnp.int32) % N
    pad_dst = N + (jnp.arange(pad, dtype=jnp.int32) % (N_PAD - N))
    src = jnp.concatenate([edge_index[0], pad_src])
    dst = jnp.concatenate([edge_index[1], pad_dst])
    src = src.reshape(NUM_TILES, CHUNKS_PER_TILE, CHUNK)
    dst = dst.reshape(NUM_TILES, CHUNKS_PER_TILE, CHUNK)
    zeros = jnp.zeros((CHUNK, D), jnp.float32)
    partials = _SC_AGG(src, dst, x, zeros)
    partials = partials.reshape(NUM_CORES, N_PAD, D)
    scale = (1.0 + eps).astype(jnp.float32).reshape(1, 1)
    return _tc_dense(x, partials, W, b.reshape(1, D), scale)


# speedup vs baseline: 3.1401x; 1.0243x over previous
import functools

import jax
import jax.numpy as jnp
from jax import lax
from jax.experimental import pallas as pl
from jax.experimental.pallas import tpu as pltpu
from jax.experimental.pallas import tpu_sc as plsc

N = 10000
E = 320000
D = 128

NUM_CORES = 2
NUM_SUBCORES = 16
NUM_TILES = NUM_CORES * NUM_SUBCORES
CHUNK = 128
CHUNKS_PER_TILE = 80
E_PAD = NUM_TILES * CHUNKS_PER_TILE * CHUNK
IDX_GRP = 40
NUM_GRPS = CHUNKS_PER_TILE // IDX_GRP
ROWS_PER_TILE = 640
N_PAD = ROWS_PER_TILE * NUM_SUBCORES
WB_BLKS = ROWS_PER_TILE // CHUNK


def _sc_aggregate():
    mesh = plsc.VectorSubcoreMesh(core_axis_name="c", subcore_axis_name="s")

    @functools.partial(
        pl.kernel,
        mesh=mesh,
        out_type=jax.ShapeDtypeStruct((NUM_CORES, NUM_SUBCORES, ROWS_PER_TILE, D),
                                      jnp.float32),
        scratch_types=[
            pltpu.VMEM((IDX_GRP, CHUNK), jnp.int32),
            pltpu.VMEM((IDX_GRP, CHUNK), jnp.int32),
            pltpu.VMEM((CHUNK, D), jnp.float32),
            pltpu.VMEM((CHUNK, D), jnp.float32),
            pltpu.VMEM_SHARED((N_PAD, D), jnp.float32),
            pltpu.SemaphoreType.DMA,
            pltpu.SemaphoreType.DMA,
            pltpu.SemaphoreType.DMA,
            pltpu.SemaphoreType.DMA,
        ],
    )
    def sc_agg(src_hbm, dst_hbm, x_hbm, zeros_hbm, out_hbm,
               src_v, dst_v, rows0, rows1, acc_sh, gs0, gs1, ss0, ss1):
        cid = lax.axis_index("c")
        sid = lax.axis_index("s")
        wid = cid * NUM_SUBCORES + sid
        row_off = pl.multiple_of(sid * ROWS_PER_TILE, 8)

        pltpu.sync_copy(zeros_hbm, rows0)
        for j in range(WB_BLKS):
            pltpu.sync_copy(rows0, acc_sh.at[pl.ds(row_off + j * CHUNK, CHUNK)])
        plsc.subcore_barrier()

        bufs = (rows0, rows1)
        gsems = (gs0, gs1)
        ssems = (ss0, ss1)

        def sg(j, k):
            pltpu.async_copy(x_hbm.at[src_v.at[j]], bufs[k], gsems[k])

        def wg(k):
            pltpu.make_async_copy(zeros_hbm, bufs[k], gsems[k]).wait()

        def st(j, k):
            pltpu.async_copy(bufs[k], acc_sh.at[dst_v.at[j]], ssems[k], add=True)

        def ws(k):
            pltpu.make_async_copy(zeros_hbm, bufs[k], ssems[k]).wait()

        def group(grp, carry):
            goff = pl.multiple_of(grp * IDX_GRP, 8)
            pltpu.sync_copy(src_hbm.at[wid, pl.ds(goff, IDX_GRP)], src_v)
            pltpu.sync_copy(dst_hbm.at[wid, pl.ds(goff, IDX_GRP)], dst_v)

            sg(0, 0)
            sg(1, 1)

            def body(g, c):
                j = 2 * g
                wg(0)
                st(j, 0)
                wg(1)
                st(j + 1, 1)
                ws(0)
                sg(j + 2, 0)
                ws(1)
                sg(j + 3, 1)
                return c

            lax.fori_loop(0, IDX_GRP // 2 - 1, body, 0)
            last = IDX_GRP - 2
            wg(0)
            st(last, 0)
            wg(1)
            st(last + 1, 1)
            ws(0)
            ws(1)
            return carry

        lax.fori_loop(0, NUM_GRPS, group, 0)

        plsc.subcore_barrier()

        for j in range(WB_BLKS):
            pltpu.sync_copy(acc_sh.at[pl.ds(row_off + j * CHUNK, CHUNK)], rows0)
            pltpu.sync_copy(rows0, out_hbm.at[cid, sid, pl.ds(j * CHUNK, CHUNK)])

    return sc_agg


_SC_AGG = _sc_aggregate()


def _tc_fn(x_ref, a_ref, w_ref, b_ref, s_ref, o_ref):
    h = s_ref[0, 0] * x_ref[...] + a_ref[0] + a_ref[1]
    o = jnp.dot(h, w_ref[...], preferred_element_type=jnp.float32) + b_ref[...]
    o_ref[...] = jnp.maximum(o, 0.0)


ROW_BLK = 2000


def _tc_dense(x, partials, W, b2, scale):
    return pl.pallas_call(
        _tc_fn,
        grid=(N // ROW_BLK,),
        in_specs=[
            pl.BlockSpec((ROW_BLK, D), lambda i: (i, 0)),
            pl.BlockSpec((NUM_CORES, ROW_BLK, D), lambda i: (0, i, 0)),
            pl.BlockSpec((D, D), lambda i: (0, 0)),
            pl.BlockSpec((1, D), lambda i: (0, 0)),
            pl.BlockSpec(memory_space=pltpu.SMEM),
        ],
        out_specs=pl.BlockSpec((ROW_BLK, D), lambda i: (i, 0)),
        out_shape=jax.ShapeDtypeStruct((N, D), jnp.float32),
    )(x, partials, W, b2, scale)


def kernel(x, edge_index, W, b, eps):
    pad = E_PAD - E
    pad_src = jnp.arange(pad, dtype=jnp.int32) % N
    pad_dst = N + (jnp.arange(pad, dtype=jnp.int32) % (N_PAD - N))
    src = jnp.concatenate([edge_index[0], pad_src])
    dst = jnp.concatenate([edge_index[1], pad_dst])
    src = src.reshape(NUM_TILES, CHUNKS_PER_TILE, CHUNK)
    dst = dst.reshape(NUM_TILES, CHUNKS_PER_TILE, CHUNK)
    zeros = jnp.zeros((CHUNK, D), jnp.float32)
    partials = _SC_AGG(src, dst, x, zeros)
    partials = partials.reshape(NUM_CORES, N_PAD, D)
    scale = (1.0 + eps).astype(jnp.float32).reshape(1, 1)
    return _tc_dense(x, partials, W, b.reshape(1, D), scale)
